# baseline (speedup 1.0000x reference)
import jax
import jax.numpy as jnp
from jax.experimental import pallas as pl
from jax.experimental.pallas import tpu as pltpu

_T = 2048
_A = 64
_NA = 256
_TC = _T - _A + 1
_OFF = _A - 1
_ROWS = 2176
_GJ = 2 * _A
_TPAD = 2112


def _u_kernel(dnp_ref, u_ref):
    for j in range(_GJ):
        u_ref[:, j, :] = dnp_ref[:, j:j + _A]


_NS = 1


def _mp_kernel(niter_ref, pt_ref, u_ref, dnt_ref, dncol_ref, out_ref,
               p_ref, fm_ref, ragg_ref):
    rows = jax.lax.broadcasted_iota(jnp.int32, (_ROWS, _NA), 0)
    valid = (rows >= _OFF) & (rows < _OFF + _TC)
    rows1 = jax.lax.broadcasted_iota(jnp.int32, (_ROWS, 1), 0)
    rowsw = jax.lax.broadcasted_iota(jnp.int32, (136, 1), 0)
    sub8 = jax.lax.broadcasted_iota(jnp.int32, (8, _NA), 0)
    lane8 = jax.lax.broadcasted_iota(jnp.int32, (8, _NA), 1)

    out_ref[...] = jnp.zeros_like(out_ref)
    for i in range(_NS):
        p_ref[i] = jnp.transpose(pt_ref[i], (1, 0))
        fm_ref[i] = jnp.dot(p_ref[i].astype(jnp.bfloat16), dnt_ref[...],
                            preferred_element_type=jnp.float32)
        ragg_ref[i] = jnp.max(jnp.where(valid, jnp.abs(fm_ref[i]), -1.0),
                              axis=1, keepdims=True)

    def body(_, carry):
        for i in range(_NS):
            ragg = ragg_ref[i]
            m = jnp.max(ragg)
            rstar = jnp.min(jnp.where(ragg == m, rows1, jnp.int32(2**30)))
            rbase = pl.multiple_of((rstar // 8) * 8, 8)
            tile = fm_ref[i, pl.ds(rbase, 8), :]
            hit = sub8 == rstar - rbase
            astar = jnp.min(jnp.where(hit & (jnp.abs(tile) == m), lane8,
                                      jnp.int32(2**30)))
            coeff = jnp.sum(jnp.where(hit & (lane8 == astar), tile, 0.0))
            t = rstar - _OFF
            base = pl.multiple_of((t // 8) * 8, 8)
            rem = t - base
            ublk = u_ref[astar]
            zu = jnp.zeros((8, _A), jnp.float32)
            ucat = jnp.concatenate([zu, ublk, zu], axis=0)
            ush = jax.lax.switch(
                rem,
                [lambda p=p: jax.lax.slice_in_dim(ucat, 8 - p, 144 - p, axis=0)
                 for p in range(8)])
            cur = p_ref[i, pl.ds(base, 136), :]
            pw = cur - coeff * ush
            p_ref[i, pl.ds(base, 136), :] = pw
            fmw = jnp.dot(pw.astype(jnp.bfloat16), dnt_ref[...],
                          preferred_element_type=jnp.float32)
            fm_ref[i, pl.ds(base, 136), :] = fmw
            validw = ((base + rowsw) >= _OFF) & ((base + rowsw) < _OFF + _TC)
            ragg_ref[i, pl.ds(base, 136), :] = jnp.where(
                validw, jnp.max(jnp.abs(fmw), axis=1, keepdims=True), -1.0)
            atom = dncol_ref[astar]
            z1 = jnp.zeros((8, 1), jnp.float32)
            acat = jnp.concatenate([z1, atom, z1], axis=0)
            ash = jax.lax.switch(
                rem,
                [lambda p=p: jax.lax.slice_in_dim(acat, 8 - p, 80 - p, axis=0)
                 for p in range(8)])
            cur2 = out_ref[i, pl.ds(base, 72), :]
            out_ref[i, pl.ds(base, 72), :] = cur2 + coeff * ash
        return carry

    jax.lax.fori_loop(0, niter_ref[0], body, 0)


def kernel(x, d, n_iterations):
    B, C, T = x.shape
    na, _, A = d.shape
    dn = d / jnp.sqrt(jnp.sum(d * d, axis=(-1, -2), keepdims=True) + 1e-8)
    dn2 = dn[:, 0, :]
    x2 = x[:, 0, :]

    xe = jnp.pad(x2, ((0, 0), (_OFF, _ROWS + A - T - _OFF)))
    pt = jnp.stack([xe[:, k:k + _ROWS] for k in range(A)], axis=1)

    dnp = jnp.pad(dn2, ((0, 0), (_OFF, A)))
    dnt = dn2.T.astype(jnp.bfloat16)

    dncol = dn2[:, :, None]
    niter = jnp.asarray(n_iterations, jnp.int32).reshape(1)

    def mp_stage(niter, pt, dnp, dnt, dncol):
        bloc = pt.shape[0]
        u = pl.pallas_call(
            _u_kernel,
            out_shape=jax.ShapeDtypeStruct((na, _GJ, A), jnp.float32),
        )(dnp)
        return pl.pallas_call(
            _mp_kernel,
            grid=(bloc // _NS,),
            in_specs=[
                pl.BlockSpec(memory_space=pltpu.SMEM),
                pl.BlockSpec((_NS, A, _ROWS), lambda b: (b, 0, 0)),
                pl.BlockSpec((na, _GJ, A), lambda b: (0, 0, 0)),
                pl.BlockSpec((A, na), lambda b: (0, 0)),
                pl.BlockSpec((na, A, 1), lambda b: (0, 0, 0)),
            ],
            out_specs=pl.BlockSpec((_NS, _TPAD, 1), lambda b: (b, 0, 0)),
            out_shape=jax.ShapeDtypeStruct((bloc, _TPAD, 1), jnp.float32),
            scratch_shapes=[pltpu.VMEM((_NS, _ROWS, _A), jnp.float32),
                            pltpu.VMEM((_NS, _ROWS, _NA), jnp.float32),
                            pltpu.VMEM((_NS, _ROWS, 1), jnp.float32)],
        )(niter, pt, u, dnt, dncol)

    devs = jax.devices()
    nd = 1
    for cand in (8, 4, 2):
        if len(devs) >= cand and B % cand == 0:
            nd = cand
            break
    if nd > 1:
        from jax.sharding import Mesh, PartitionSpec as P
        try:
            from jax.experimental.shard_map import shard_map
        except ImportError:
            from jax import shard_map
        import numpy as np
        mesh = Mesh(np.array(devs[:nd]), ("b",))
        mp = shard_map(mp_stage, mesh=mesh,
                       in_specs=(P(), P("b", None, None), P(None, None),
                                 P(None, None), P(None, None, None)),
                       out_specs=P("b", None, None), check_rep=False)
    else:
        mp = mp_stage
    recon = mp(niter, pt, dnp, dnt, dncol)

    return recon[:, :T, 0][:, None, :]


# speedup vs baseline: 1.0611x; 1.0611x over previous
import jax
import jax.numpy as jnp
from jax.experimental import pallas as pl
from jax.experimental.pallas import tpu as pltpu

_T = 2048
_A = 64
_NA = 256
_TC = _T - _A + 1
_OFF = _A - 1
_ROWS = 2176
_GJ = 2 * _A
_TPAD = 2112


def _u_kernel(dnp_ref, u_ref):
    for j in range(_GJ):
        u_ref[:, j, :] = dnp_ref[:, j:j + _A]


_NS = 1


def _mp_kernel(niter_ref, pt_ref, u_ref, dnt_ref, dncol_ref, out_ref,
               p_ref, fm_ref, ragg_ref):
    rows = jax.lax.broadcasted_iota(jnp.int32, (_ROWS, _NA), 0)
    valid = (rows >= _OFF) & (rows < _OFF + _TC)
    rows1 = jax.lax.broadcasted_iota(jnp.int32, (_ROWS, 1), 0)
    rowsw = jax.lax.broadcasted_iota(jnp.int32, (136, 1), 0)
    sub8 = jax.lax.broadcasted_iota(jnp.int32, (8, _NA), 0)
    lane8 = jax.lax.broadcasted_iota(jnp.int32, (8, _NA), 1)

    out_ref[...] = jnp.zeros_like(out_ref)
    for i in range(_NS):
        p_ref[i] = jnp.transpose(pt_ref[i], (1, 0))
        fm_ref[i] = jnp.dot(p_ref[i].astype(jnp.bfloat16), dnt_ref[...],
                            preferred_element_type=jnp.float32)
        ragg_ref[i] = jnp.max(jnp.where(valid, jnp.abs(fm_ref[i]), -1.0),
                              axis=1, keepdims=True)

    def body(_, carry):
        for i in range(_NS):
            ragg = ragg_ref[i]
            m = jnp.max(ragg)
            rstar = jnp.min(jnp.where(ragg == m, rows1, jnp.int32(2**30)))
            rbase = pl.multiple_of((rstar // 8) * 8, 8)
            tile = fm_ref[i, pl.ds(rbase, 8), :]
            hit = sub8 == rstar - rbase
            astar = jnp.min(jnp.where(hit & (jnp.abs(tile) == m), lane8,
                                      jnp.int32(2**30)))
            coeff = jnp.sum(jnp.where(hit & (lane8 == astar), tile, 0.0))
            t = rstar - _OFF
            base = pl.multiple_of((t // 8) * 8, 8)
            rem = t - base
            ublk = u_ref[astar]
            zu = jnp.zeros((8, _A), jnp.float32)
            ucat = jnp.concatenate([zu, ublk, zu], axis=0)
            ush = jax.lax.switch(
                rem,
                [lambda p=p: jax.lax.slice_in_dim(ucat, 8 - p, 144 - p, axis=0)
                 for p in range(8)])
            cur = p_ref[i, pl.ds(base, 136), :]
            pw = cur - coeff * ush
            p_ref[i, pl.ds(base, 136), :] = pw
            fmw = jnp.dot(pw.astype(jnp.bfloat16), dnt_ref[...],
                          preferred_element_type=jnp.float32)
            fm_ref[i, pl.ds(base, 136), :] = fmw
            validw = ((base + rowsw) >= _OFF) & ((base + rowsw) < _OFF + _TC)
            ragg_ref[i, pl.ds(base, 136), :] = jnp.where(
                validw, jnp.max(jnp.abs(fmw), axis=1, keepdims=True), -1.0)
            atom = dncol_ref[astar]
            z1 = jnp.zeros((8, 1), jnp.float32)
            acat = jnp.concatenate([z1, atom, z1], axis=0)
            ash = jax.lax.switch(
                rem,
                [lambda p=p: jax.lax.slice_in_dim(acat, 8 - p, 80 - p, axis=0)
                 for p in range(8)])
            cur2 = out_ref[i, pl.ds(base, 72), :]
            out_ref[i, pl.ds(base, 72), :] = cur2 + coeff * ash
        return carry

    jax.lax.fori_loop(0, niter_ref[0], body, 0)


def _pipeline(x, d, niter):
    B, C, T = x.shape
    na, _, A = d.shape
    dn = d / jnp.sqrt(jnp.sum(d * d, axis=(-1, -2), keepdims=True) + 1e-8)
    dn2 = dn[:, 0, :]
    x2 = x[:, 0, :]

    xe = jnp.pad(x2, ((0, 0), (_OFF, _ROWS + A - T - _OFF)))
    pt = jnp.stack([xe[:, k:k + _ROWS] for k in range(A)], axis=1)

    dnp = jnp.pad(dn2, ((0, 0), (_OFF, A)))
    dnt = dn2.T.astype(jnp.bfloat16)
    dncol = dn2[:, :, None]

    u = pl.pallas_call(
        _u_kernel,
        out_shape=jax.ShapeDtypeStruct((na, _GJ, A), jnp.float32),
    )(dnp)
    recon = pl.pallas_call(
        _mp_kernel,
        grid=(B // _NS,),
        in_specs=[
            pl.BlockSpec(memory_space=pltpu.SMEM),
            pl.BlockSpec((_NS, A, _ROWS), lambda b: (b, 0, 0)),
            pl.BlockSpec((na, _GJ, A), lambda b: (0, 0, 0)),
            pl.BlockSpec((A, na), lambda b: (0, 0)),
            pl.BlockSpec((na, A, 1), lambda b: (0, 0, 0)),
        ],
        out_specs=pl.BlockSpec((_NS, _TPAD, 1), lambda b: (b, 0, 0)),
        out_shape=jax.ShapeDtypeStruct((B, _TPAD, 1), jnp.float32),
        scratch_shapes=[pltpu.VMEM((_NS, _ROWS, _A), jnp.float32),
                        pltpu.VMEM((_NS, _ROWS, _NA), jnp.float32),
                        pltpu.VMEM((_NS, _ROWS, 1), jnp.float32)],
    )(niter, pt, u, dnt, dncol)
    return recon[:, :T, 0][:, None, :]


def kernel(x, d, n_iterations):
    B = x.shape[0]
    niter = jnp.asarray(n_iterations, jnp.int32).reshape(1)

    devs = jax.devices()
    nd = 1
    for cand in (8, 4, 2):
        if len(devs) >= cand and B % cand == 0:
            nd = cand
            break
    if nd > 1:
        from jax.sharding import Mesh, PartitionSpec as P
        try:
            from jax.experimental.shard_map import shard_map
        except ImportError:
            from jax import shard_map
        import numpy as np
        mesh = Mesh(np.array(devs[:nd]), ("b",))
        f = shard_map(_pipeline, mesh=mesh,
                      in_specs=(P("b", None, None), P(None, None, None), P()),
                      out_specs=P("b", None, None), check_rep=False)
    else:
        f = _pipeline
    return f(x, d, niter)


# speedup vs baseline: 1.5820x; 1.4909x over previous
import jax
import jax.numpy as jnp
from jax.experimental import pallas as pl
from jax.experimental.pallas import tpu as pltpu

_T = 2048
_A = 64
_NA = 256
_TC = _T - _A + 1
_OFF = _A - 1
_ROWS = 2176
_GJ = 2 * _A
_TPAD = 2112


def _u_kernel(dnp_ref, u_ref):
    for j in range(_GJ):
        u_ref[:, j, :] = dnp_ref[:, j:j + _A]


_NS = 1


def _mp_kernel(niter_ref, pt_ref, u_ref, dnt_ref, dncol_ref, out_ref,
               p_ref, fm_ref, ragg_ref):
    rows = jax.lax.broadcasted_iota(jnp.int32, (_ROWS, _NA), 0)
    valid = (rows >= _OFF) & (rows < _OFF + _TC)
    rows1 = jax.lax.broadcasted_iota(jnp.int32, (_ROWS, 1), 0)
    rowsw = jax.lax.broadcasted_iota(jnp.int32, (136, 1), 0)
    sub8 = jax.lax.broadcasted_iota(jnp.int32, (8, _NA), 0)
    lane8 = jax.lax.broadcasted_iota(jnp.int32, (8, _NA), 1)

    out_ref[...] = jnp.zeros_like(out_ref)
    for i in range(_NS):
        p_ref[i] = jnp.transpose(pt_ref[i], (1, 0))
        fm_ref[i] = jnp.dot(p_ref[i].astype(jnp.bfloat16), dnt_ref[...],
                            preferred_element_type=jnp.float32)
        ragg_ref[i] = jnp.max(jnp.where(valid, jnp.abs(fm_ref[i]), -1.0),
                              axis=1, keepdims=True)

    def body(_, carry):
        for i in range(_NS):
            ragg = ragg_ref[i]
            m = jnp.max(ragg)
            rstar = jnp.min(jnp.where(ragg == m, rows1, jnp.int32(2**30)))
            rbase = pl.multiple_of((rstar // 8) * 8, 8)
            tile = fm_ref[i, pl.ds(rbase, 8), :]
            hit = sub8 == rstar - rbase
            astar = jnp.min(jnp.where(hit & (jnp.abs(tile) == m), lane8,
                                      jnp.int32(2**30)))
            coeff = jnp.sum(jnp.where(hit & (lane8 == astar), tile, 0.0))
            t = rstar - _OFF
            base = pl.multiple_of((t // 8) * 8, 8)
            rem = t - base
            ublk = u_ref[astar]
            zu = jnp.zeros((8, _A), jnp.float32)
            ucat = jnp.concatenate([zu, ublk, zu], axis=0)
            ush = jax.lax.switch(
                rem,
                [lambda p=p: jax.lax.slice_in_dim(ucat, 8 - p, 144 - p, axis=0)
                 for p in range(8)])
            cur = p_ref[i, pl.ds(base, 136), :]
            pw = cur - coeff * ush
            p_ref[i, pl.ds(base, 136), :] = pw
            fmw = jnp.dot(pw.astype(jnp.bfloat16), dnt_ref[...],
                          preferred_element_type=jnp.float32)
            fm_ref[i, pl.ds(base, 136), :] = fmw
            validw = ((base + rowsw) >= _OFF) & ((base + rowsw) < _OFF + _TC)
            ragg_ref[i, pl.ds(base, 136), :] = jnp.where(
                validw, jnp.max(jnp.abs(fmw), axis=1, keepdims=True), -1.0)
            atom = dncol_ref[astar]
            z1 = jnp.zeros((8, 1), jnp.float32)
            acat = jnp.concatenate([z1, atom, z1], axis=0)
            ash = jax.lax.switch(
                rem,
                [lambda p=p: jax.lax.slice_in_dim(acat, 8 - p, 80 - p, axis=0)
                 for p in range(8)])
            cur2 = out_ref[i, pl.ds(base, 72), :]
            out_ref[i, pl.ds(base, 72), :] = cur2 + coeff * ash
        return carry

    jax.lax.fori_loop(0, niter_ref[0], body, 0)


def _pipeline(x, d, niter):
    B, C, T = x.shape
    na, _, A = d.shape
    dn = d / jnp.sqrt(jnp.sum(d * d, axis=(-1, -2), keepdims=True) + 1e-8)
    dn2 = dn[:, 0, :]
    x2 = x[:, 0, :]

    xe = jnp.pad(x2, ((0, 0), (_OFF, _ROWS + A - T - _OFF)))
    pt = jnp.stack([xe[:, k:k + _ROWS] for k in range(A)], axis=1)

    dnp = jnp.pad(dn2, ((0, 0), (_OFF, A)))
    dnt = dn2.T.astype(jnp.bfloat16)
    dncol = dn2[:, :, None]

    u = pl.pallas_call(
        _u_kernel,
        out_shape=jax.ShapeDtypeStruct((na, _GJ, A), jnp.float32),
    )(dnp)
    recon = pl.pallas_call(
        _mp_kernel,
        grid=(B // _NS,),
        in_specs=[
            pl.BlockSpec(memory_space=pltpu.SMEM),
            pl.BlockSpec((_NS, A, _ROWS), lambda b: (b, 0, 0)),
            pl.BlockSpec((na, _GJ, A), lambda b: (0, 0, 0)),
            pl.BlockSpec((A, na), lambda b: (0, 0)),
            pl.BlockSpec((na, A, 1), lambda b: (0, 0, 0)),
        ],
        out_specs=pl.BlockSpec((_NS, _TPAD, 1), lambda b: (b, 0, 0)),
        out_shape=jax.ShapeDtypeStruct((B, _TPAD, 1), jnp.float32),
        scratch_shapes=[pltpu.VMEM((_NS, _ROWS, _A), jnp.float32),
                        pltpu.VMEM((_NS, _ROWS, _NA), jnp.float32),
                        pltpu.VMEM((_NS, _ROWS, 1), jnp.float32)],
    )(niter, pt, u, dnt, dncol)
    return recon[:, :T, 0][:, None, :]


def kernel(x, d, n_iterations):
    B = x.shape[0]
    niter = jnp.asarray(n_iterations, jnp.int32).reshape(1)

    return _pipeline(x, d, niter)
